# Initial kernel scaffold; baseline (speedup 1.0000x reference)
import jax
import jax.numpy as jnp
from jax.experimental import pallas as pl


def kernel(z_e, codebook):
    raise NotImplementedError("write your pallas kernel here")


# speedup vs baseline: 1.0469x; 1.0469x over previous
import functools

import jax
import jax.numpy as jnp
from jax import lax
from jax.experimental import pallas as pl
from jax.experimental.pallas import tpu as pltpu
from jax.experimental.pallas import tpu_sc as plsc

_B = 16384
_V = 8192
_D = 32
_BM = 256


_H = _V // 2


def _dist_argmin_body(z_ref, s1_ref, cb_ref, idx_ref, loss_ref):
    i = pl.program_id(0)
    z = z_ref[...]
    cb = cb_ref[...]
    s1 = s1_ref[...]
    s2 = jnp.sum(cb * cb, axis=1)
    m = lax.dot_general(z, cb, (((1,), (1,)), ((), ())),
                        preferred_element_type=jnp.float32)
    d = (s1 - 2.0 * m) + s2[None, :]
    ids = lax.broadcasted_iota(jnp.int32, d.shape, 1)
    dA, dB = d[:, :_H], d[:, _H:]
    minA = jnp.min(dA, axis=1, keepdims=True)
    minB = jnp.min(dB, axis=1, keepdims=True)
    idxA = jnp.min(jnp.where(dA == minA, ids[:, :_H], _V), axis=1)
    idxB = jnp.min(jnp.where(dB == minB, ids[:, _H:], _V), axis=1)
    bA = minA.astype(jnp.bfloat16).astype(jnp.float32)
    take_b = minB < bA
    idx = jnp.where(take_b[:, 0], idxB, idxA)
    chosen = jnp.where(take_b, minB, minA)
    idx_ref[...] = idx

    @pl.when(i == 0)
    def _():
        loss_ref[0, 0] = 0.0

    loss_ref[0, 0] += jnp.sum(chosen)


def _dist_argmin(z_e, s1, codebook):
    return pl.pallas_call(
        _dist_argmin_body,
        grid=(_B // _BM,),
        in_specs=[
            pl.BlockSpec((_BM, _D), lambda i: (i, 0)),
            pl.BlockSpec((_BM, 1), lambda i: (i, 0)),
            pl.BlockSpec((_V, _D), lambda i: (0, 0)),
        ],
        out_specs=[
            pl.BlockSpec((_BM,), lambda i: (i,)),
            pl.BlockSpec(memory_space=pltpu.SMEM),
        ],
        out_shape=[
            jax.ShapeDtypeStruct((_B,), jnp.int32),
            jax.ShapeDtypeStruct((1, 1), jnp.float32),
        ],
    )(z_e, s1, codebook)


_NW = 32
_BPW = _B // _NW


def _gather_body(cb_hbm, idx_hbm, out_hbm, idx_v, rows_v, sem):
    wid = lax.axis_index("s") * 2 + lax.axis_index("c")
    base = wid * _BPW
    pltpu.sync_copy(idx_hbm.at[pl.ds(base, _BPW)], idx_v)
    pltpu.async_copy(cb_hbm.at[idx_v], rows_v, sem).wait()
    pltpu.sync_copy(rows_v, out_hbm.at[pl.ds(base, _BPW)])


_sc_gather = pl.kernel(
    _gather_body,
    out_type=jax.ShapeDtypeStruct((_B, _D), jnp.float32),
    mesh=plsc.VectorSubcoreMesh(core_axis_name="c", subcore_axis_name="s"),
    scratch_types=[
        pltpu.VMEM((_BPW,), jnp.int32),
        pltpu.VMEM((_BPW, _D), jnp.float32),
        pltpu.SemaphoreType.DMA,
    ],
    compiler_params=pltpu.CompilerParams(use_tc_tiling_on_sc=False),
)


def kernel(z_e, codebook):
    s1 = jnp.sum(z_e ** 2, axis=1, keepdims=True)
    indices, loss_acc = _dist_argmin(z_e, s1, codebook)
    z_q = _sc_gather(codebook, indices)
    vq_loss = jnp.reshape(loss_acc * (0.25 / (_B * _D)), ())
    return (z_q, vq_loss, indices)


# speedup vs baseline: 1.5628x; 1.4928x over previous
import functools

import jax
import jax.numpy as jnp
from jax import lax
from jax.experimental import pallas as pl
from jax.experimental.pallas import tpu as pltpu
from jax.experimental.pallas import tpu_sc as plsc

_B = 16384
_V = 8192
_D = 32
_BM = 1024


_H = _V // 2


def _dist_argmin_body(z_ref, s1_ref, cb_ref, s2_ref, ids_ref, idx_ref, loss_ref):
    i = pl.program_id(0)
    z = z_ref[...]
    cb = cb_ref[...]
    s1 = s1_ref[...]
    s2 = s2_ref[...]
    z2 = z + z
    m2 = lax.dot_general(z2, cb, (((1,), (1,)), ((), ())),
                         preferred_element_type=jnp.float32)
    d = (s1 - m2) + s2[None, :]
    ids = ids_ref[...][None, :]
    dA, dB = d[:, :_H], d[:, _H:]
    minA = jnp.min(dA, axis=1, keepdims=True)
    minB = jnp.min(dB, axis=1, keepdims=True)
    fV = jnp.float32(_V)
    idxA = jnp.min(jnp.where(dA == minA, jnp.broadcast_to(ids[:, :_H], dA.shape), fV), axis=1)
    idxB = jnp.min(jnp.where(dB == minB, jnp.broadcast_to(ids[:, _H:], dB.shape), fV), axis=1)
    bA = minA.astype(jnp.bfloat16).astype(jnp.float32)
    take_b = minB < bA
    idx = jnp.where(take_b[:, 0], idxB, idxA).astype(jnp.int32)
    chosen = jnp.where(take_b, minB, minA)
    idx_ref[...] = idx

    @pl.when(i == 0)
    def _():
        loss_ref[0, 0] = 0.0

    loss_ref[0, 0] += jnp.sum(chosen)


def _dist_argmin(z_e, s1, codebook, s2, ids):
    return pl.pallas_call(
        _dist_argmin_body,
        grid=(_B // _BM,),
        in_specs=[
            pl.BlockSpec((_BM, _D), lambda i: (i, 0)),
            pl.BlockSpec((_BM, 1), lambda i: (i, 0)),
            pl.BlockSpec((_V, _D), lambda i: (0, 0)),
            pl.BlockSpec((_V,), lambda i: (0,)),
            pl.BlockSpec((_V,), lambda i: (0,)),
        ],
        out_specs=[
            pl.BlockSpec((_BM,), lambda i: (i,)),
            pl.BlockSpec(memory_space=pltpu.SMEM),
        ],
        out_shape=[
            jax.ShapeDtypeStruct((_B,), jnp.int32),
            jax.ShapeDtypeStruct((1, 1), jnp.float32),
        ],
    )(z_e, s1, codebook, s2, ids)


_NW = 32
_BPW = _B // _NW


def _gather_body(cb_hbm, idx_hbm, out_hbm, idx_v, rows_v, sem):
    wid = lax.axis_index("s") * 2 + lax.axis_index("c")
    base = wid * _BPW
    pltpu.sync_copy(idx_hbm.at[pl.ds(base, _BPW)], idx_v)
    pltpu.async_copy(cb_hbm.at[idx_v], rows_v, sem).wait()
    pltpu.sync_copy(rows_v, out_hbm.at[pl.ds(base, _BPW)])


_sc_gather = pl.kernel(
    _gather_body,
    out_type=jax.ShapeDtypeStruct((_B, _D), jnp.float32),
    mesh=plsc.VectorSubcoreMesh(core_axis_name="c", subcore_axis_name="s"),
    scratch_types=[
        pltpu.VMEM((_BPW,), jnp.int32),
        pltpu.VMEM((_BPW, _D), jnp.float32),
        pltpu.SemaphoreType.DMA,
    ],
    compiler_params=pltpu.CompilerParams(use_tc_tiling_on_sc=False),
)


def kernel(z_e, codebook):
    s1 = jnp.sum(z_e ** 2, axis=1, keepdims=True)
    s2 = jnp.sum(codebook ** 2, axis=1)
    ids = jnp.arange(_V, dtype=jnp.float32)
    indices, loss_acc = _dist_argmin(z_e, s1, codebook, s2, ids)
    z_q = _sc_gather(codebook, indices)
    vq_loss = jnp.reshape(loss_acc * (0.25 / (_B * _D)), ())
    return (z_q, vq_loss, indices)
